# Initial kernel scaffold; baseline (speedup 1.0000x reference)
import jax
import jax.numpy as jnp
from jax.experimental import pallas as pl


def kernel(x, keys, values, params):
    raise NotImplementedError("write your pallas kernel here")


# speedup vs baseline: 1.7956x; 1.7956x over previous
import functools

import numpy as np
import jax
import jax.numpy as jnp
from jax import lax
from jax.experimental import pallas as pl
from jax.experimental.pallas import tpu as pltpu
from jax.experimental.pallas import tpu_sc as plsc

B, C, NTOK, DK, DV, P, NHEADS = 1024, 8, 1, 64, 64, 8192, 2
HD = DK // NHEADS
N = B * NTOK
EPS = 1e-5
PC = 2048
NJ = P // PC


def _ln(x, g, b):
    m = jnp.mean(x, axis=-1, keepdims=True)
    v = jnp.mean((x - m) ** 2, axis=-1, keepdims=True)
    return (x - m) / jnp.sqrt(v + EPS) * g + b


def _bf16(x):
    return x.astype(jnp.bfloat16).astype(jnp.float32)


def _attn_kernel(xt_ref, ln1g_ref, ln1b_ref, win_ref, bin_ref, wout_ref,
                 bout_ref, ln2g_ref, ln2b_ref, w1_ref, b1_ref, w2_ref,
                 b2_ref, wdec_ref, bdec_ref, f_ref):
    ln1g = ln1g_ref[...]
    ln1b = ln1b_ref[...]
    win = win_ref[...]
    bin_ = bin_ref[...]

    q = []
    k = []
    v = []
    flat = []
    for c in range(C):
        xc = xt_ref[c]
        flat.append(xc)
        h = _ln(xc, ln1g, ln1b)
        qkv = jnp.dot(h, win, preferred_element_type=jnp.float32) + bin_
        q.append(_bf16(qkv[:, :DK]))
        k.append(_bf16(qkv[:, DK:2 * DK]))
        v.append(_bf16(qkv[:, 2 * DK:]))

    dd = lax.broadcasted_iota(jnp.int32, (C * DK, NHEADS * C), 0)
    cc = lax.broadcasted_iota(jnp.int32, (C * DK, NHEADS * C), 1)
    hsel = jnp.where((dd // DK == cc % C) & ((dd % DK) // HD == cc // C),
                     1.0, 0.0).astype(jnp.float32)

    wout = wout_ref[...]
    bout = bout_ref[...]
    ln2g = ln2g_ref[...]
    ln2b = ln2b_ref[...]
    w1 = w1_ref[...]
    b1 = b1_ref[...]
    w2 = w2_ref[...]
    b2 = b2_ref[...]
    wdec = wdec_ref[...]
    bdec = bdec_ref[...]

    for l in range(C):
        ql = q[l]
        prod = jnp.concatenate([ql * k[m] for m in range(C)], axis=1)
        s = jnp.dot(prod, hsel, preferred_element_type=jnp.float32,
                    precision=jax.lax.Precision.HIGHEST)
        s = s / np.sqrt(HD)
        o_heads = []
        for h in range(NHEADS):
            sh = s[:, h * C:(h + 1) * C]
            sh = sh - jnp.max(sh, axis=1, keepdims=True)
            e = jnp.exp(sh)
            a = _bf16(e / jnp.sum(e, axis=1, keepdims=True))
            acc = a[:, 0:1] * v[0][:, h * HD:(h + 1) * HD]
            for m in range(1, C):
                acc = acc + a[:, m:m + 1] * v[m][:, h * HD:(h + 1) * HD]
            o_heads.append(acc)
        o = jnp.concatenate(o_heads, axis=1)
        o = jnp.dot(o, wout, preferred_element_type=jnp.float32) + bout
        mha = o + flat[l]
        g = _ln(mha, ln2g, ln2b)
        g = jnp.dot(g, w1, preferred_element_type=jnp.float32) + b1
        g = 0.5 * g * (1.0 + lax.erf(g / np.sqrt(2.0)))
        g = jnp.dot(g, w2, preferred_element_type=jnp.float32) + b2
        g = g + mha
        f_ref[l] = jnp.dot(g, wdec, preferred_element_type=jnp.float32) + bdec


def _dist_kernel(f_ref, keys_ref, idx_ref, best_ref):
    c = pl.program_id(0)
    j = pl.program_id(1)
    f = f_ref[0]
    kb = keys_ref[0]
    mmt = lax.dot_general(kb, f, (((1,), (1,)), ((), ())),
                          preferred_element_type=jnp.float32)
    k2 = jnp.sum(kb * kb, axis=1, keepdims=True)
    s = 2.0 * mmt - k2
    m = jnp.max(s, axis=0, keepdims=True)
    row = lax.broadcasted_iota(jnp.int32, (PC, N), 0)
    li = jnp.min(jnp.where(s == m, row, jnp.int32(2 ** 30)),
                 axis=0, keepdims=True)
    gi = li + (j * PC + c * P)

    @pl.when(j == 0)
    def _init():
        best_ref[...] = m
        idx_ref[0] = gi

    @pl.when(j > 0)
    def _update():
        upd = m > best_ref[...]
        best_ref[...] = jnp.where(upd, m, best_ref[...])
        idx_ref[0] = jnp.where(upd, gi, idx_ref[0])


def _compute_f(xt, params):
    ops = [
        params['ln1_g'].reshape(1, DK), params['ln1_b'].reshape(1, DK),
        params['in_proj_w'].T, params['in_proj_b'].reshape(1, 3 * DK),
        params['out_proj_w'].T, params['out_proj_b'].reshape(1, DK),
        params['ln2_g'].reshape(1, DK), params['ln2_b'].reshape(1, DK),
        params['ffn1_w'].T, params['ffn1_b'].reshape(1, DK),
        params['ffn2_w'].T, params['ffn2_b'].reshape(1, DK),
        params['dec_w'].T, params['dec_b'].reshape(1, DK),
    ]
    return pl.pallas_call(
        _attn_kernel,
        out_shape=jax.ShapeDtypeStruct((C, N, DK), jnp.float32),
    )(xt, *ops)


def _compute_idx(f, keys):
    return pl.pallas_call(
        _dist_kernel,
        grid=(C, NJ),
        in_specs=[
            pl.BlockSpec((1, N, DK), lambda c, j: (c, 0, 0)),
            pl.BlockSpec((1, PC, DK), lambda c, j: (c, j, 0)),
        ],
        out_specs=pl.BlockSpec((1, 1, N), lambda c, j: (c, 0, 0)),
        out_shape=jax.ShapeDtypeStruct((C, 1, N), jnp.int32),
        scratch_shapes=[pltpu.VMEM((1, N), jnp.float32)],
    )(f, keys)


def _gather_values(table, gidx):
    info = plsc.get_sparse_core_info()
    nw = info.num_cores * info.num_subcores
    bpw = (C * N) // nw
    mesh = plsc.VectorSubcoreMesh(core_axis_name="c", subcore_axis_name="s")

    @functools.partial(
        pl.kernel, mesh=mesh,
        compiler_params=pltpu.CompilerParams(use_tc_tiling_on_sc=False),
        out_type=jax.ShapeDtypeStruct((C * N, DV), jnp.float32),
        scratch_types=[
            pltpu.VMEM((bpw,), jnp.int32),
            pltpu.VMEM((bpw, DV), jnp.float32),
            pltpu.SemaphoreType.DMA,
        ],
    )
    def gk(table_hbm, idx_hbm, out_hbm, idx_v, rows_v, sem):
        wid = lax.axis_index("s") * info.num_cores + lax.axis_index("c")
        base = wid * bpw
        pltpu.sync_copy(idx_hbm.at[pl.ds(base, bpw)], idx_v)
        pltpu.async_copy(table_hbm.at[idx_v], rows_v, sem).wait()
        pltpu.sync_copy(rows_v, out_hbm.at[pl.ds(base, bpw)])

    return gk(table, gidx)


def kernel(x, keys, values, params):
    xt = jnp.transpose(x.astype(jnp.float32), (1, 0, 2, 3)).reshape(C, N, DK)
    f = _compute_f(xt, params)
    idx = _compute_idx(f, keys)
    gidx = jnp.transpose(idx[:, 0, :], (1, 0)).reshape(C * N)
    table = values.reshape(C * P, DV)
    rows = _gather_values(table, gidx)
    return rows.reshape(B, C, NTOK, DV)


# speedup vs baseline: 1.9043x; 1.0605x over previous
import functools

import numpy as np
import jax
import jax.numpy as jnp
from jax import lax
from jax.experimental import pallas as pl
from jax.experimental.pallas import tpu as pltpu
from jax.experimental.pallas import tpu_sc as plsc

B, C, NTOK, DK, DV, P, NHEADS = 1024, 8, 1, 64, 64, 8192, 2
HD = DK // NHEADS
N = B * NTOK
EPS = 1e-5
PC = 2048
NJ = P // PC


def _ln(x, g, b):
    m = jnp.mean(x, axis=-1, keepdims=True)
    v = jnp.mean((x - m) ** 2, axis=-1, keepdims=True)
    return (x - m) / jnp.sqrt(v + EPS) * g + b


def _bf16(x):
    return x.astype(jnp.bfloat16).astype(jnp.float32)


def _attn_kernel(xt_ref, ln1g_ref, ln1b_ref, win_ref, bin_ref, wout_ref,
                 bout_ref, ln2g_ref, ln2b_ref, w1_ref, b1_ref, w2_ref,
                 b2_ref, wdec_ref, bdec_ref, f_ref):
    ln1g = ln1g_ref[...]
    ln1b = ln1b_ref[...]
    win = win_ref[...]
    bin_ = bin_ref[...]

    q = []
    k = []
    v = []
    flat = []
    for c in range(C):
        xc = xt_ref[:, c * DK:(c + 1) * DK]
        flat.append(xc)
        h = _ln(xc, ln1g, ln1b)
        qkv = jnp.dot(h, win, preferred_element_type=jnp.float32) + bin_
        q.append(_bf16(qkv[:, :DK]))
        k.append(_bf16(qkv[:, DK:2 * DK]))
        v.append(_bf16(qkv[:, 2 * DK:]))

    dd = lax.broadcasted_iota(jnp.int32, (C * DK, NHEADS * C), 0)
    cc = lax.broadcasted_iota(jnp.int32, (C * DK, NHEADS * C), 1)
    hsel = jnp.where((dd // DK == cc % C) & ((dd % DK) // HD == cc // C),
                     1.0, 0.0).astype(jnp.float32)

    wout = wout_ref[...]
    bout = bout_ref[...]
    ln2g = ln2g_ref[...]
    ln2b = ln2b_ref[...]
    w1 = w1_ref[...]
    b1 = b1_ref[...]
    w2 = w2_ref[...]
    b2 = b2_ref[...]
    wdec = wdec_ref[...]
    bdec = bdec_ref[...]

    for l in range(C):
        ql = q[l]
        prod = jnp.concatenate([ql * k[m] for m in range(C)], axis=1)
        s = jnp.dot(prod, hsel, preferred_element_type=jnp.float32,
                    precision=jax.lax.Precision.HIGHEST)
        s = s / np.sqrt(HD)
        o_heads = []
        for h in range(NHEADS):
            sh = s[:, h * C:(h + 1) * C]
            sh = sh - jnp.max(sh, axis=1, keepdims=True)
            e = jnp.exp(sh)
            a = _bf16(e / jnp.sum(e, axis=1, keepdims=True))
            acc = a[:, 0:1] * v[0][:, h * HD:(h + 1) * HD]
            for m in range(1, C):
                acc = acc + a[:, m:m + 1] * v[m][:, h * HD:(h + 1) * HD]
            o_heads.append(acc)
        o = jnp.concatenate(o_heads, axis=1)
        o = jnp.dot(o, wout, preferred_element_type=jnp.float32) + bout
        mha = o + flat[l]
        g = _ln(mha, ln2g, ln2b)
        g = jnp.dot(g, w1, preferred_element_type=jnp.float32) + b1
        g = 0.5 * g * (1.0 + lax.erf(g / np.sqrt(2.0)))
        g = jnp.dot(g, w2, preferred_element_type=jnp.float32) + b2
        g = g + mha
        f_ref[l] = jnp.dot(g, wdec, preferred_element_type=jnp.float32) + bdec


def _dist_kernel(f_ref, keys_ref, idx_ref, best_ref):
    c = pl.program_id(0)
    j = pl.program_id(1)
    f = f_ref[0]
    kb = keys_ref[0]
    mmt = lax.dot_general(kb, f, (((1,), (1,)), ((), ())),
                          preferred_element_type=jnp.float32)
    k2 = jnp.sum(kb * kb, axis=1, keepdims=True) * 0.25
    s = mmt - k2
    m = jnp.max(s, axis=0, keepdims=True)
    row = lax.broadcasted_iota(jnp.int32, (PC, N), 0)
    li = jnp.min(jnp.where(s == m, row, jnp.int32(2 ** 30)),
                 axis=0, keepdims=True)
    gi = li + (j * PC + c * P)

    @pl.when(j == 0)
    def _init():
        best_ref[...] = m
        idx_ref[0] = gi

    @pl.when(j > 0)
    def _update():
        upd = m > best_ref[...]
        best_ref[...] = jnp.where(upd, m, best_ref[...])
        idx_ref[0] = jnp.where(upd, gi, idx_ref[0])


def _compute_f(xt, params):
    ops = [
        params['ln1_g'].reshape(1, DK), params['ln1_b'].reshape(1, DK),
        params['in_proj_w'].T, params['in_proj_b'].reshape(1, 3 * DK),
        params['out_proj_w'].T, params['out_proj_b'].reshape(1, DK),
        params['ln2_g'].reshape(1, DK), params['ln2_b'].reshape(1, DK),
        params['ffn1_w'].T, params['ffn1_b'].reshape(1, DK),
        params['ffn2_w'].T, params['ffn2_b'].reshape(1, DK),
        params['dec_w'].T, params['dec_b'].reshape(1, DK),
    ]
    return pl.pallas_call(
        _attn_kernel,
        out_shape=jax.ShapeDtypeStruct((C, N, DK), jnp.float32),
    )(xt, *ops)


def _compute_idx(f, keys):
    return pl.pallas_call(
        _dist_kernel,
        grid=(C, NJ),
        in_specs=[
            pl.BlockSpec((1, N, DK), lambda c, j: (c, 0, 0)),
            pl.BlockSpec((1, PC, DK), lambda c, j: (c, j, 0)),
        ],
        out_specs=pl.BlockSpec((1, 1, N), lambda c, j: (c, 0, 0)),
        out_shape=jax.ShapeDtypeStruct((C, 1, N), jnp.int32),
        scratch_shapes=[pltpu.VMEM((1, N), jnp.float32)],
    )(f, keys * 2.0)


def _gather_values(table, gidx):
    info = plsc.get_sparse_core_info()
    nw = info.num_cores * info.num_subcores
    bpw = (C * N) // nw
    wpc = N // bpw
    mesh = plsc.VectorSubcoreMesh(core_axis_name="c", subcore_axis_name="s")

    @functools.partial(
        pl.kernel, mesh=mesh,
        compiler_params=pltpu.CompilerParams(use_tc_tiling_on_sc=False),
        out_type=jax.ShapeDtypeStruct((N, C, DV), jnp.float32),
        scratch_types=[
            pltpu.VMEM((bpw,), jnp.int32),
            pltpu.VMEM((bpw, DV), jnp.float32),
            pltpu.SemaphoreType.DMA,
        ],
    )
    def gk(table_hbm, idx_hbm, out_hbm, idx_v, rows_v, sem):
        wid = lax.axis_index("s") * info.num_cores + lax.axis_index("c")
        cb = wid // wpc
        b0 = (wid % wpc) * bpw
        pltpu.sync_copy(idx_hbm.at[pl.ds(wid * bpw, bpw)], idx_v)
        pltpu.async_copy(table_hbm.at[idx_v], rows_v, sem).wait()
        pltpu.sync_copy(rows_v, out_hbm.at[pl.ds(b0, bpw), cb])

    return gk(table, gidx)


def kernel(x, keys, values, params):
    xt = x.astype(jnp.float32).reshape(N, C * DK)
    f = _compute_f(xt, params)
    idx = _compute_idx(f, keys)
    gidx = idx.reshape(C * N)
    table = values.reshape(C * P, DV)
    rows = _gather_values(table, gidx)
    return rows.reshape(B, C, NTOK, DV)


# speedup vs baseline: 1.9256x; 1.0112x over previous
import functools

import numpy as np
import jax
import jax.numpy as jnp
from jax import lax
from jax.experimental import pallas as pl
from jax.experimental.pallas import tpu as pltpu
from jax.experimental.pallas import tpu_sc as plsc

B, C, NTOK, DK, DV, P, NHEADS = 1024, 8, 1, 64, 64, 8192, 2
HD = DK // NHEADS
N = B * NTOK
EPS = 1e-5
PC = 4096
NJ = P // PC


def _ln(x, g, b):
    m = jnp.mean(x, axis=-1, keepdims=True)
    v = jnp.mean((x - m) ** 2, axis=-1, keepdims=True)
    return (x - m) / jnp.sqrt(v + EPS) * g + b


def _bf16(x):
    return x.astype(jnp.bfloat16).astype(jnp.float32)


def _attn_kernel(xt_ref, ln1g_ref, ln1b_ref, win_ref, bin_ref, wout_ref,
                 bout_ref, ln2g_ref, ln2b_ref, w1_ref, b1_ref, w2_ref,
                 b2_ref, wdec_ref, bdec_ref, f_ref):
    ln1g = ln1g_ref[...]
    ln1b = ln1b_ref[...]
    win = win_ref[...]
    bin_ = bin_ref[...]

    q = []
    k = []
    v = []
    flat = []
    for c in range(C):
        xc = xt_ref[:, c * DK:(c + 1) * DK]
        flat.append(xc)
        h = _ln(xc, ln1g, ln1b)
        qkv = jnp.dot(h, win, preferred_element_type=jnp.float32) + bin_
        q.append(_bf16(qkv[:, :DK]))
        k.append(_bf16(qkv[:, DK:2 * DK]))
        v.append(_bf16(qkv[:, 2 * DK:]))

    dd = lax.broadcasted_iota(jnp.int32, (C * DK, NHEADS * C), 0)
    cc = lax.broadcasted_iota(jnp.int32, (C * DK, NHEADS * C), 1)
    hsel = jnp.where((dd // DK == cc % C) & ((dd % DK) // HD == cc // C),
                     1.0, 0.0).astype(jnp.float32)

    wout = wout_ref[...]
    bout = bout_ref[...]
    ln2g = ln2g_ref[...]
    ln2b = ln2b_ref[...]
    w1 = w1_ref[...]
    b1 = b1_ref[...]
    w2 = w2_ref[...]
    b2 = b2_ref[...]
    wdec = wdec_ref[...]
    bdec = bdec_ref[...]

    for l in range(C):
        ql = q[l]
        prod = jnp.concatenate([ql * k[m] for m in range(C)], axis=1)
        s = jnp.dot(prod, hsel, preferred_element_type=jnp.float32,
                    precision=jax.lax.Precision.HIGHEST)
        s = s / np.sqrt(HD)
        o_heads = []
        for h in range(NHEADS):
            sh = s[:, h * C:(h + 1) * C]
            sh = sh - jnp.max(sh, axis=1, keepdims=True)
            e = jnp.exp(sh)
            a = _bf16(e / jnp.sum(e, axis=1, keepdims=True))
            acc = a[:, 0:1] * v[0][:, h * HD:(h + 1) * HD]
            for m in range(1, C):
                acc = acc + a[:, m:m + 1] * v[m][:, h * HD:(h + 1) * HD]
            o_heads.append(acc)
        o = jnp.concatenate(o_heads, axis=1)
        o = jnp.dot(o, wout, preferred_element_type=jnp.float32) + bout
        mha = o + flat[l]
        g = _ln(mha, ln2g, ln2b)
        g = jnp.dot(g, w1, preferred_element_type=jnp.float32) + b1
        g = 0.5 * g * (1.0 + lax.erf(g / np.sqrt(2.0)))
        g = jnp.dot(g, w2, preferred_element_type=jnp.float32) + b2
        g = g + mha
        f_ref[l] = jnp.dot(g, wdec, preferred_element_type=jnp.float32) + bdec


def _dist_kernel(f_ref, keys_ref, idx_ref, best_ref):
    c = pl.program_id(0)
    j = pl.program_id(1)
    f = f_ref[0]
    kb = keys_ref[0]
    mmt = lax.dot_general(kb, f, (((1,), (1,)), ((), ())),
                          preferred_element_type=jnp.float32)
    k2 = jnp.sum(kb * kb, axis=1, keepdims=True) * 0.25
    s = mmt - k2
    m = jnp.max(s, axis=0, keepdims=True)
    row = lax.broadcasted_iota(jnp.int32, (PC, N), 0)
    li = jnp.min(jnp.where(s == m, row, jnp.int32(2 ** 30)),
                 axis=0, keepdims=True)
    gi = li + (j * PC + c * P)

    @pl.when(j == 0)
    def _init():
        best_ref[...] = m
        idx_ref[0] = gi

    @pl.when(j > 0)
    def _update():
        upd = m > best_ref[...]
        best_ref[...] = jnp.where(upd, m, best_ref[...])
        idx_ref[0] = jnp.where(upd, gi, idx_ref[0])


def _compute_f(xt, params):
    ops = [
        params['ln1_g'].reshape(1, DK), params['ln1_b'].reshape(1, DK),
        params['in_proj_w'].T, params['in_proj_b'].reshape(1, 3 * DK),
        params['out_proj_w'].T, params['out_proj_b'].reshape(1, DK),
        params['ln2_g'].reshape(1, DK), params['ln2_b'].reshape(1, DK),
        params['ffn1_w'].T, params['ffn1_b'].reshape(1, DK),
        params['ffn2_w'].T, params['ffn2_b'].reshape(1, DK),
        params['dec_w'].T, params['dec_b'].reshape(1, DK),
    ]
    return pl.pallas_call(
        _attn_kernel,
        out_shape=jax.ShapeDtypeStruct((C, N, DK), jnp.float32),
    )(xt, *ops)


def _compute_idx(f, keys):
    return pl.pallas_call(
        _dist_kernel,
        grid=(C, NJ),
        in_specs=[
            pl.BlockSpec((1, N, DK), lambda c, j: (c, 0, 0)),
            pl.BlockSpec((1, PC, DK), lambda c, j: (c, j, 0)),
        ],
        out_specs=pl.BlockSpec((1, 1, N), lambda c, j: (c, 0, 0)),
        out_shape=jax.ShapeDtypeStruct((C, 1, N), jnp.int32),
        scratch_shapes=[pltpu.VMEM((1, N), jnp.float32)],
    )(f, keys * 2.0)


def _gather_values(table, gidx):
    info = plsc.get_sparse_core_info()
    nw = info.num_cores * info.num_subcores
    bpw = (C * N) // nw
    wpc = N // bpw
    mesh = plsc.VectorSubcoreMesh(core_axis_name="c", subcore_axis_name="s")

    @functools.partial(
        pl.kernel, mesh=mesh,
        compiler_params=pltpu.CompilerParams(use_tc_tiling_on_sc=False),
        out_type=jax.ShapeDtypeStruct((N, C, DV), jnp.float32),
        scratch_types=[
            pltpu.VMEM((bpw,), jnp.int32),
            pltpu.VMEM((bpw, DV), jnp.float32),
            pltpu.SemaphoreType.DMA,
        ],
    )
    def gk(table_hbm, idx_hbm, out_hbm, idx_v, rows_v, sem):
        wid = lax.axis_index("s") * info.num_cores + lax.axis_index("c")
        cb = wid // wpc
        b0 = (wid % wpc) * bpw
        pltpu.sync_copy(idx_hbm.at[pl.ds(wid * bpw, bpw)], idx_v)
        pltpu.async_copy(table_hbm.at[idx_v], rows_v, sem).wait()
        pltpu.sync_copy(rows_v, out_hbm.at[pl.ds(b0, bpw), cb])

    return gk(table, gidx)


def kernel(x, keys, values, params):
    xt = x.astype(jnp.float32).reshape(N, C * DK)
    f = _compute_f(xt, params)
    idx = _compute_idx(f, keys)
    gidx = idx.reshape(C * N)
    table = values.reshape(C * P, DV)
    rows = _gather_values(table, gidx)
    return rows.reshape(B, C, NTOK, DV)
